# baseline (speedup 1.0000x reference)
import functools

import jax
import jax.numpy as jnp
from jax import lax
from jax.experimental import pallas as pl
from jax.experimental.pallas import tpu as pltpu
from jax.experimental.pallas import tpu_sc as plsc

NL = 10000
ND = 4096
NLP = 10240
NC_NODES = NLP + ND
FD = 128

NCORES = 2
NSUB = 16
NW = NCORES * NSUB

EL = 320000
ED = 131072
E = EL + ED
CHUNK = 32
EBLK = 160
EPT = 14400
EPAD = EPT * NW
NBLKE = EPT // EBLK
NCH = EBLK // CHUNK

ROWS_PER_TILE = NC_NODES // NSUB
BN = 512
NBLK = NC_NODES // BN
LSM_BLOCKS = NLP // BN

_MESH = plsc.VectorSubcoreMesh(core_axis_name="c", subcore_axis_name="s")


def _zero_vmem_2d(ref, nrow):
    def body(j, _):
        for k in range(ref.shape[1] // 16):
            ref[j, pl.ds(k * 16, 16)] = jnp.zeros((16,), jnp.float32)
        return 0
    lax.fori_loop(0, nrow, body, 0)


@functools.partial(
    pl.kernel,
    out_type=jax.ShapeDtypeStruct((NCORES, NC_NODES), jnp.float32),
    mesh=_MESH,
    scratch_types=[
        pltpu.VMEM_SHARED((NC_NODES,), jnp.float32),
        pltpu.VMEM((EBLK,), jnp.int32),
        pltpu.VMEM((EBLK,), jnp.float32),
        pltpu.VMEM((ROWS_PER_TILE,), jnp.float32),
    ],
)
def _deg_kernel(col_hbm, w_hbm, out_hbm, acc, col_v, w_v, zbuf):
    cid = lax.axis_index("c")
    sid = lax.axis_index("s")
    wid = cid * NSUB + sid

    def zb(j, _):
        zbuf[pl.ds(j * 16, 16)] = jnp.zeros((16,), jnp.float32)
        return 0
    lax.fori_loop(0, ROWS_PER_TILE // 16, zb, 0)
    pltpu.sync_copy(zbuf, acc.at[pl.ds(sid * ROWS_PER_TILE, ROWS_PER_TILE)])
    plsc.subcore_barrier()

    def blk_body(t, _):
        base = wid * (EPAD // NW) + t * EBLK
        pltpu.sync_copy(col_hbm.at[pl.ds(base, EBLK)], col_v)
        pltpu.sync_copy(w_hbm.at[pl.ds(base, EBLK)], w_v)
        pltpu.sync_copy(w_v, acc.at[col_v], add=True)
        return 0
    lax.fori_loop(0, (EPAD // NW) // EBLK, blk_body, 0)
    plsc.subcore_barrier()

    pltpu.sync_copy(acc.at[pl.ds(sid * ROWS_PER_TILE, ROWS_PER_TILE)],
                    out_hbm.at[cid, pl.ds(sid * ROWS_PER_TILE, ROWS_PER_TILE)])


@functools.partial(
    pl.kernel,
    out_type=jax.ShapeDtypeStruct((NCORES, NC_NODES, FD), jnp.float32),
    mesh=_MESH,
    scratch_types=[
        pltpu.VMEM_SHARED((NC_NODES, FD), jnp.float32),
        pltpu.VMEM((CHUNK, FD), jnp.float32),
        pltpu.VMEM((CHUNK, FD), jnp.float32),
        pltpu.VMEM((2 * NCH, CHUNK), jnp.int32),
        pltpu.VMEM((2 * NCH, CHUNK), jnp.int32),
        pltpu.VMEM((EBLK,), jnp.float32),
        pltpu.VMEM((EBLK,), jnp.float32),
        pltpu.SemaphoreType.DMA,
        pltpu.SemaphoreType.DMA,
        pltpu.SemaphoreType.DMA,
        pltpu.SemaphoreType.DMA,
        pltpu.SemaphoreType.DMA,
        pltpu.SemaphoreType.DMA,
    ],
)
def _msg_kernel(xs_hbm, packed_hbm, w3_hbm, out_hbm,
                acc, rows_a, rows_b, idx_a, idx_b, wb_a, wb_b,
                gs_a, gs_b, ss_a, ss_b, is_a, is_b):
    cid = lax.axis_index("c")
    sid = lax.axis_index("s")
    wid = cid * NSUB + sid
    bufs = (rows_a, rows_b)
    gsems = (gs_a, gs_b)
    ssems = (ss_a, ss_b)
    ibufs = (idx_a, idx_b)
    wbufs = (wb_a, wb_b)
    isems = (is_a, is_b)

    _zero_vmem_2d(rows_a, CHUNK)
    r0 = sid * ROWS_PER_TILE
    done = 0
    while done < ROWS_PER_TILE:
        n = min(CHUNK, ROWS_PER_TILE - done)
        pltpu.sync_copy(rows_a.at[pl.ds(0, n)], acc.at[pl.ds(r0 + done, n)])
        done += n
    plsc.subcore_barrier()

    def scale(buf, wb, c):
        def scale_grp(jj, _):
            wv = wb[pl.ds(c * CHUNK + jj * 16, 16)]
            for i in range(16):
                wj = wv[i]
                j = jj * 16 + i
                for k in range(FD // 16):
                    buf[j, pl.ds(k * 16, 16)] = buf[j, pl.ds(k * 16, 16)] * wj
            return 0
        lax.fori_loop(0, CHUNK // 16, scale_grp, 0)

    def idx_start(t2, blk, p):
        b = wid * NBLKE + 2 * t2 + blk
        d0 = pltpu.async_copy(packed_hbm.at[b], ibufs[p], isems[p])
        d1 = pltpu.async_copy(w3_hbm.at[b], wbufs[p], isems[p])
        return d0, d1

    def idx_wait(p):
        pltpu.make_async_copy(packed_hbm.at[0], ibufs[p], isems[p]).wait()
        pltpu.make_async_copy(w3_hbm.at[0], wbufs[p], isems[p]).wait()

    def process_pair(t2):
        nc2 = 2 * NCH

        def src(c):
            return ibufs[c // NCH], wbufs[c // NCH], c % NCH

        gd = [None, None]
        sd = [None] * nc2
        ib0, _, _ = src(0)
        gd[0] = pltpu.async_copy(xs_hbm.at[ib0.at[0]], bufs[0], gsems[0])
        for c in range(nc2):
            p = c % 2
            ib, wb, cc = src(c)
            if c + 1 < nc2:
                q = 1 - p
                if c >= 1:
                    sd[c - 1].wait()
                if c + 1 == NCH:
                    idx_wait(1)
                ib1, _, cc1 = src(c + 1)
                gd[q] = pltpu.async_copy(xs_hbm.at[ib1.at[cc1]],
                                         bufs[q], gsems[q])
            gd[p].wait()
            scale(bufs[p], wb, cc)
            sd[c] = pltpu.async_copy(bufs[p], acc.at[ib.at[NCH + cc]],
                                     ssems[p], add=True)
        sd[nc2 - 1].wait()

    for d in idx_start(0, 0, 0):
        d.wait()

    def pair_body(t2, _):
        idx_start(t2, 1, 1)
        process_pair(t2)

        @pl.when(t2 + 1 < NBLKE // 2)
        def _():
            idx_start(t2 + 1, 0, 0)
            idx_wait(0)
        return 0
    lax.fori_loop(0, NBLKE // 2, pair_body, 0)
    plsc.subcore_barrier()

    pltpu.sync_copy(acc.at[pl.ds(r0, ROWS_PER_TILE)],
                    out_hbm.at[cid, pl.ds(r0, ROWS_PER_TILE)])


def _dinv_block(deg_blk):
    deg = deg_blk[0] + deg_blk[1] + 1.0
    return jnp.where(deg > 0, lax.rsqrt(deg), 0.0)


def _xs_body(x_ref, w_ref, deg_ref, o_ref):
    dinv = _dinv_block(deg_ref[...])
    o_ref[...] = dinv[:, None] * jnp.dot(x_ref[...], w_ref[0],
                                         preferred_element_type=jnp.float32)


def _region(i):
    return jnp.where(i < LSM_BLOCKS, 0, 1)


def _xs_call(x, w_stack, deg_parts):
    return pl.pallas_call(
        _xs_body,
        grid=(NBLK,),
        in_specs=[
            pl.BlockSpec((BN, FD), lambda i: (i, 0)),
            pl.BlockSpec((1, FD, FD), lambda i: (_region(i), 0, 0)),
            pl.BlockSpec((NCORES, BN), lambda i: (0, i)),
        ],
        out_specs=pl.BlockSpec((BN, FD), lambda i: (i, 0)),
        out_shape=jax.ShapeDtypeStruct((NC_NODES, FD), jnp.float32),
    )(x, w_stack, deg_parts)


def _bnorm_relu(t, g, be):
    m = jnp.mean(t, axis=1, keepdims=True)
    v = jnp.mean((t - m) ** 2, axis=1, keepdims=True)
    return jax.nn.relu((t - m) / jnp.sqrt(v + 1e-5) * g[:, None] + be[:, None])


def _post1_xs2_body(acc_ref, xs_ref, deg_ref, b_ref, g_ref, be_ref, w2_ref,
                    x1_ref, xs2_ref):
    dinv = _dinv_block(deg_ref[...])
    t = dinv[:, None] * (acc_ref[0] + acc_ref[1] + xs_ref[...]) + b_ref[0, 0]
    x1 = _bnorm_relu(t, g_ref[0], be_ref[0])
    x1_ref[...] = x1
    xs2_ref[...] = dinv[:, None] * jnp.dot(
        x1, w2_ref[0], preferred_element_type=jnp.float32)


def _post1_xs2_call(acc_parts, xs, deg_parts, b_stack, g, be, w2_stack):
    return pl.pallas_call(
        _post1_xs2_body,
        grid=(NBLK,),
        in_specs=[
            pl.BlockSpec((NCORES, BN, FD), lambda i: (0, i, 0)),
            pl.BlockSpec((BN, FD), lambda i: (i, 0)),
            pl.BlockSpec((NCORES, BN), lambda i: (0, i)),
            pl.BlockSpec((1, 1, FD), lambda i: (_region(i), 0, 0)),
            pl.BlockSpec((1, BN), lambda i: (0, i)),
            pl.BlockSpec((1, BN), lambda i: (0, i)),
            pl.BlockSpec((1, FD, FD), lambda i: (_region(i), 0, 0)),
        ],
        out_specs=[pl.BlockSpec((BN, FD), lambda i: (i, 0)),
                   pl.BlockSpec((BN, FD), lambda i: (i, 0))],
        out_shape=[jax.ShapeDtypeStruct((NC_NODES, FD), jnp.float32),
                   jax.ShapeDtypeStruct((NC_NODES, FD), jnp.float32)],
    )(acc_parts, xs, deg_parts, b_stack, g, be, w2_stack)


def _post2_att_body(acc_ref, xs_ref, deg_ref, b_ref, g_ref, be_ref, res_ref,
                    wa_ref, ga_ref, bea_ref, wt_ref, xo_ref, p_ref):
    dinv = _dinv_block(deg_ref[...])
    t = dinv[:, None] * (acc_ref[0] + acc_ref[1] + xs_ref[...]) + b_ref[0, 0]
    z = _bnorm_relu(t, g_ref[0], be_ref[0])
    x2 = jax.nn.relu(res_ref[...] + z)
    a = jnp.tanh(jnp.dot(x2, wa_ref[0], preferred_element_type=jnp.float32))
    e = jnp.exp(a - jnp.max(a, axis=1, keepdims=True))
    alpha = e / jnp.sum(e, axis=1, keepdims=True)
    xo = _bnorm_relu(alpha * x2, ga_ref[0], bea_ref[0])
    xo_ref[...] = xo
    p_ref[...] = jnp.dot(xo, wt_ref[...], preferred_element_type=jnp.float32)


def _post2_att_call(acc_parts, xs, deg_parts, b_stack, g, be, res,
                    wa_stack, ga, bea, weight):
    return pl.pallas_call(
        _post2_att_body,
        grid=(NBLK,),
        in_specs=[
            pl.BlockSpec((NCORES, BN, FD), lambda i: (0, i, 0)),
            pl.BlockSpec((BN, FD), lambda i: (i, 0)),
            pl.BlockSpec((NCORES, BN), lambda i: (0, i)),
            pl.BlockSpec((1, 1, FD), lambda i: (_region(i), 0, 0)),
            pl.BlockSpec((1, BN), lambda i: (0, i)),
            pl.BlockSpec((1, BN), lambda i: (0, i)),
            pl.BlockSpec((BN, FD), lambda i: (i, 0)),
            pl.BlockSpec((1, FD, FD), lambda i: (_region(i), 0, 0)),
            pl.BlockSpec((1, BN), lambda i: (0, i)),
            pl.BlockSpec((1, BN), lambda i: (0, i)),
            pl.BlockSpec((FD, FD), lambda i: (0, 0)),
        ],
        out_specs=[pl.BlockSpec((BN, FD), lambda i: (i, 0)),
                   pl.BlockSpec((BN, FD), lambda i: (i, 0))],
        out_shape=[jax.ShapeDtypeStruct((NC_NODES, FD), jnp.float32),
                   jax.ShapeDtypeStruct((NC_NODES, FD), jnp.float32)],
    )(acc_parts, xs, deg_parts, b_stack, g, be, res, wa_stack, ga, bea, weight)


def _final_body(p_ref, yo_ref, o_ref):
    s = lax.dot_general(p_ref[...], yo_ref[...], (((1,), (1,)), ((), ())),
                        preferred_element_type=jnp.float32)
    o_ref[...] = jax.nn.sigmoid(s)


def _final_call(p_pad, yo):
    return pl.pallas_call(
        _final_body,
        grid=(NLP // BN, ND // BN),
        in_specs=[
            pl.BlockSpec((BN, FD), lambda i, j: (i, 0)),
            pl.BlockSpec((BN, FD), lambda i, j: (j, 0)),
        ],
        out_specs=pl.BlockSpec((BN, BN), lambda i, j: (i, j)),
        out_shape=jax.ShapeDtypeStruct((NLP, ND), jnp.float32),
    )(p_pad, yo)


def _pad_rows(a, n):
    return jnp.concatenate([a, jnp.zeros((n - a.shape[0],) + a.shape[1:],
                                         a.dtype)], axis=0)


def _combine_vec(al, ad):
    return jnp.concatenate([al, jnp.zeros((NLP - NL,), al.dtype), ad]
                           )[None, :]


def kernel(x_l, x_d, lsm_edges, lsm_edge_weight, dsm_edges, dsm_edge_weight,
           W_lsm1, b_lsm1, W_lsm2, b_lsm2, W_dsm1, b_dsm1, W_dsm2, b_dsm2,
           Wa_x, Wa_y, weight,
           g1x, be1x, g2x, be2x, gx, bex,
           g1y, be1y, g2y, be2y, gy, bey):
    npad = EPAD - E
    pad_idx = (jnp.arange(npad, dtype=jnp.int32) % ND)
    row = jnp.concatenate([lsm_edges[0], dsm_edges[0] + NLP, pad_idx])
    col = jnp.concatenate([lsm_edges[1], dsm_edges[1] + NLP, pad_idx])
    w = jnp.concatenate([lsm_edge_weight, dsm_edge_weight,
                         jnp.zeros((npad,), jnp.float32)])
    nbt = EPAD // EBLK
    packed = jnp.concatenate(
        [row.reshape(nbt, NCH, CHUNK), col.reshape(nbt, NCH, CHUNK)], axis=1)
    w3 = w.reshape(nbt, EBLK)
    xcomb = jnp.concatenate([x_l, jnp.zeros((NLP - NL, FD), jnp.float32),
                             x_d], axis=0)

    W1 = jnp.stack([W_lsm1, W_dsm1])
    W2 = jnp.stack([W_lsm2, W_dsm2])
    Wa = jnp.stack([Wa_x, Wa_y])
    b1 = jnp.stack([b_lsm1, b_dsm1])[:, None, :]
    b2 = jnp.stack([b_lsm2, b_dsm2])[:, None, :]
    g1 = _combine_vec(g1x, g1y)
    be1 = _combine_vec(be1x, be1y)
    g2 = _combine_vec(g2x, g2y)
    be2 = _combine_vec(be2x, be2y)
    ga = _combine_vec(gx, gy)
    bea = _combine_vec(bex, bey)

    deg_parts = _deg_kernel(col, w)

    xs1 = _xs_call(xcomb, W1, deg_parts)
    acc1 = _msg_kernel(xs1, packed, w3)
    x1, xs2 = _post1_xs2_call(acc1, xs1, deg_parts, b1, g1, be1, W2)

    acc2 = _msg_kernel(xs2, packed, w3)
    xo_comb, p = _post2_att_call(acc2, xs2, deg_parts, b2, g2, be2, x1,
                                 Wa, ga, bea, weight)
    yo = xo_comb[NLP:]
    xo = xo_comb[:NL]

    out = _final_call(p[:NLP], yo)[:NL]
    return (xo, yo, out)


# speedup vs baseline: 1.0904x; 1.0904x over previous
import functools

import jax
import jax.numpy as jnp
from jax import lax
from jax.experimental import pallas as pl
from jax.experimental.pallas import tpu as pltpu
from jax.experimental.pallas import tpu_sc as plsc

NL = 10000
ND = 4096
NLP = 10240
NC_NODES = NLP + ND
FD = 128

NCORES = 2
NSUB = 16
NW = NCORES * NSUB

EL = 320000
ED = 131072
E = EL + ED
CHUNK = 48
EBLK = 240
EPT = 14400
EPAD = EPT * NW
NBLKE = EPT // EBLK
NCH = EBLK // CHUNK
NACC = NL + ND
ACC_PER_TILE = 881

ROWS_PER_TILE = NC_NODES // NSUB
BN = 512
NBLK = NC_NODES // BN
LSM_BLOCKS = NLP // BN

_MESH = plsc.VectorSubcoreMesh(core_axis_name="c", subcore_axis_name="s")


def _zero_vmem_2d(ref, nrow):
    def body(j, _):
        for k in range(ref.shape[1] // 16):
            ref[j, pl.ds(k * 16, 16)] = jnp.zeros((16,), jnp.float32)
        return 0
    lax.fori_loop(0, nrow, body, 0)


@functools.partial(
    pl.kernel,
    out_type=jax.ShapeDtypeStruct((NCORES, NC_NODES), jnp.float32),
    mesh=_MESH,
    scratch_types=[
        pltpu.VMEM_SHARED((NC_NODES,), jnp.float32),
        pltpu.VMEM((EBLK,), jnp.int32),
        pltpu.VMEM((EBLK,), jnp.float32),
        pltpu.VMEM((ROWS_PER_TILE,), jnp.float32),
    ],
)
def _deg_kernel(col_hbm, w_hbm, out_hbm, acc, col_v, w_v, zbuf):
    cid = lax.axis_index("c")
    sid = lax.axis_index("s")
    wid = cid * NSUB + sid

    def zb(j, _):
        zbuf[pl.ds(j * 16, 16)] = jnp.zeros((16,), jnp.float32)
        return 0
    lax.fori_loop(0, ROWS_PER_TILE // 16, zb, 0)
    pltpu.sync_copy(zbuf, acc.at[pl.ds(sid * ROWS_PER_TILE, ROWS_PER_TILE)])
    plsc.subcore_barrier()

    def blk_body(t, _):
        base = wid * (EPAD // NW) + t * EBLK
        pltpu.sync_copy(col_hbm.at[pl.ds(base, EBLK)], col_v)
        pltpu.sync_copy(w_hbm.at[pl.ds(base, EBLK)], w_v)
        pltpu.sync_copy(w_v, acc.at[col_v], add=True)
        return 0
    lax.fori_loop(0, (EPAD // NW) // EBLK, blk_body, 0)
    plsc.subcore_barrier()

    pltpu.sync_copy(acc.at[pl.ds(sid * ROWS_PER_TILE, ROWS_PER_TILE)],
                    out_hbm.at[cid, pl.ds(sid * ROWS_PER_TILE, ROWS_PER_TILE)])


@functools.partial(
    pl.kernel,
    out_type=jax.ShapeDtypeStruct((NCORES, NC_NODES, FD), jnp.float32),
    mesh=_MESH,
    scratch_types=[
        pltpu.VMEM_SHARED((NACC, FD), jnp.float32),
        pltpu.VMEM((CHUNK, FD), jnp.float32),
        pltpu.VMEM((CHUNK, FD), jnp.float32),
        pltpu.VMEM((2 * NCH, CHUNK), jnp.int32),
        pltpu.VMEM((2 * NCH, CHUNK), jnp.int32),
        pltpu.VMEM((EBLK,), jnp.float32),
        pltpu.VMEM((EBLK,), jnp.float32),
        pltpu.SemaphoreType.DMA,
        pltpu.SemaphoreType.DMA,
        pltpu.SemaphoreType.DMA,
        pltpu.SemaphoreType.DMA,
        pltpu.SemaphoreType.DMA,
        pltpu.SemaphoreType.DMA,
    ],
)
def _msg_kernel(xs_hbm, packed_hbm, w3_hbm, out_hbm,
                acc, rows_a, rows_b, idx_a, idx_b, wb_a, wb_b,
                gs_a, gs_b, ss_a, ss_b, is_a, is_b):
    cid = lax.axis_index("c")
    sid = lax.axis_index("s")
    wid = cid * NSUB + sid
    bufs = (rows_a, rows_b)
    gsems = (gs_a, gs_b)
    ssems = (ss_a, ss_b)
    ibufs = (idx_a, idx_b)
    wbufs = (wb_a, wb_b)
    isems = (is_a, is_b)

    _zero_vmem_2d(rows_a, CHUNK)
    z0 = sid * ACC_PER_TILE
    done = 0
    while done < ACC_PER_TILE:
        n = min(CHUNK, ACC_PER_TILE - done)
        pltpu.sync_copy(rows_a.at[pl.ds(0, n)], acc.at[pl.ds(z0 + done, n)])
        done += n
    plsc.subcore_barrier()

    def scale(buf, wb, c):
        def scale_grp(jj, _):
            wv = wb[pl.ds(c * CHUNK + jj * 16, 16)]
            for i in range(16):
                wj = wv[i]
                j = jj * 16 + i
                for k in range(FD // 16):
                    buf[j, pl.ds(k * 16, 16)] = buf[j, pl.ds(k * 16, 16)] * wj
            return 0
        lax.fori_loop(0, CHUNK // 16, scale_grp, 0)

    def idx_start(t2, blk, p):
        b = wid * NBLKE + 2 * t2 + blk
        d0 = pltpu.async_copy(packed_hbm.at[b], ibufs[p], isems[p])
        d1 = pltpu.async_copy(w3_hbm.at[b], wbufs[p], isems[p])
        return d0, d1

    def idx_wait(p):
        pltpu.make_async_copy(packed_hbm.at[0], ibufs[p], isems[p]).wait()
        pltpu.make_async_copy(w3_hbm.at[0], wbufs[p], isems[p]).wait()

    def process_pair(t2):
        nc2 = 2 * NCH

        def src(c):
            return ibufs[c // NCH], wbufs[c // NCH], c % NCH

        gd = [None, None]
        sd = [None] * nc2
        ib0, _, _ = src(0)
        gd[0] = pltpu.async_copy(xs_hbm.at[ib0.at[0]], bufs[0], gsems[0])
        for c in range(nc2):
            p = c % 2
            ib, wb, cc = src(c)
            if c + 1 < nc2:
                q = 1 - p
                if c >= 1:
                    sd[c - 1].wait()
                if c + 1 == NCH:
                    idx_wait(1)
                ib1, _, cc1 = src(c + 1)
                gd[q] = pltpu.async_copy(xs_hbm.at[ib1.at[cc1]],
                                         bufs[q], gsems[q])
            gd[p].wait()
            scale(bufs[p], wb, cc)
            sd[c] = pltpu.async_copy(bufs[p], acc.at[ib.at[NCH + cc]],
                                     ssems[p], add=True)
        sd[nc2 - 1].wait()

    for d in idx_start(0, 0, 0):
        d.wait()

    def pair_body(t2, _):
        idx_start(t2, 1, 1)
        process_pair(t2)

        @pl.when(t2 + 1 < NBLKE // 2)
        def _():
            idx_start(t2 + 1, 0, 0)
            idx_wait(0)
        return 0
    lax.fori_loop(0, NBLKE // 2, pair_body, 0)
    plsc.subcore_barrier()

    for s in range(NSUB):
        p0 = s * ROWS_PER_TILE
        p1 = p0 + ROWS_PER_TILE
        if p1 <= NL:
            @pl.when(sid == s)
            def _(p0=p0):
                pltpu.sync_copy(acc.at[pl.ds(p0, ROWS_PER_TILE)],
                                out_hbm.at[cid, pl.ds(p0, ROWS_PER_TILE)])
        elif p0 < NL:
            n1 = NL - p0
            n2 = p1 - NLP
            @pl.when(sid == s)
            def _(p0=p0, n1=n1, n2=n2):
                pltpu.sync_copy(acc.at[pl.ds(p0, n1)],
                                out_hbm.at[cid, pl.ds(p0, n1)])
                pltpu.sync_copy(acc.at[pl.ds(NL, n2)],
                                out_hbm.at[cid, pl.ds(NLP, n2)])
        else:
            @pl.when(sid == s)
            def _(p0=p0):
                pltpu.sync_copy(acc.at[pl.ds(p0 - (NLP - NL), ROWS_PER_TILE)],
                                out_hbm.at[cid, pl.ds(p0, ROWS_PER_TILE)])


def _dinv_block(deg_blk):
    deg = deg_blk[0] + deg_blk[1] + 1.0
    return jnp.where(deg > 0, lax.rsqrt(deg), 0.0)


def _xs_body(x_ref, w_ref, deg_ref, o_ref):
    dinv = _dinv_block(deg_ref[...])
    o_ref[...] = dinv[:, None] * jnp.dot(x_ref[...], w_ref[0],
                                         preferred_element_type=jnp.float32)


def _region(i):
    return jnp.where(i < LSM_BLOCKS, 0, 1)


def _xs_call(x, w_stack, deg_parts):
    return pl.pallas_call(
        _xs_body,
        grid=(NBLK,),
        in_specs=[
            pl.BlockSpec((BN, FD), lambda i: (i, 0)),
            pl.BlockSpec((1, FD, FD), lambda i: (_region(i), 0, 0)),
            pl.BlockSpec((NCORES, BN), lambda i: (0, i)),
        ],
        out_specs=pl.BlockSpec((BN, FD), lambda i: (i, 0)),
        out_shape=jax.ShapeDtypeStruct((NC_NODES, FD), jnp.float32),
    )(x, w_stack, deg_parts)


def _bnorm_relu(t, g, be):
    m = jnp.mean(t, axis=1, keepdims=True)
    v = jnp.mean((t - m) ** 2, axis=1, keepdims=True)
    return jax.nn.relu((t - m) / jnp.sqrt(v + 1e-5) * g[:, None] + be[:, None])


def _post1_xs2_body(acc_ref, xs_ref, deg_ref, b_ref, g_ref, be_ref, w2_ref,
                    x1_ref, xs2_ref):
    dinv = _dinv_block(deg_ref[...])
    t = dinv[:, None] * (acc_ref[0] + acc_ref[1] + xs_ref[...]) + b_ref[0, 0]
    x1 = _bnorm_relu(t, g_ref[0], be_ref[0])
    x1_ref[...] = x1
    xs2_ref[...] = dinv[:, None] * jnp.dot(
        x1, w2_ref[0], preferred_element_type=jnp.float32)


def _post1_xs2_call(acc_parts, xs, deg_parts, b_stack, g, be, w2_stack):
    return pl.pallas_call(
        _post1_xs2_body,
        grid=(NBLK,),
        in_specs=[
            pl.BlockSpec((NCORES, BN, FD), lambda i: (0, i, 0)),
            pl.BlockSpec((BN, FD), lambda i: (i, 0)),
            pl.BlockSpec((NCORES, BN), lambda i: (0, i)),
            pl.BlockSpec((1, 1, FD), lambda i: (_region(i), 0, 0)),
            pl.BlockSpec((1, BN), lambda i: (0, i)),
            pl.BlockSpec((1, BN), lambda i: (0, i)),
            pl.BlockSpec((1, FD, FD), lambda i: (_region(i), 0, 0)),
        ],
        out_specs=[pl.BlockSpec((BN, FD), lambda i: (i, 0)),
                   pl.BlockSpec((BN, FD), lambda i: (i, 0))],
        out_shape=[jax.ShapeDtypeStruct((NC_NODES, FD), jnp.float32),
                   jax.ShapeDtypeStruct((NC_NODES, FD), jnp.float32)],
    )(acc_parts, xs, deg_parts, b_stack, g, be, w2_stack)


def _post2_att_body(acc_ref, xs_ref, deg_ref, b_ref, g_ref, be_ref, res_ref,
                    wa_ref, ga_ref, bea_ref, wt_ref, xo_ref, p_ref):
    dinv = _dinv_block(deg_ref[...])
    t = dinv[:, None] * (acc_ref[0] + acc_ref[1] + xs_ref[...]) + b_ref[0, 0]
    z = _bnorm_relu(t, g_ref[0], be_ref[0])
    x2 = jax.nn.relu(res_ref[...] + z)
    a = jnp.tanh(jnp.dot(x2, wa_ref[0], preferred_element_type=jnp.float32))
    e = jnp.exp(a - jnp.max(a, axis=1, keepdims=True))
    alpha = e / jnp.sum(e, axis=1, keepdims=True)
    xo = _bnorm_relu(alpha * x2, ga_ref[0], bea_ref[0])
    xo_ref[...] = xo
    p_ref[...] = jnp.dot(xo, wt_ref[...], preferred_element_type=jnp.float32)


def _post2_att_call(acc_parts, xs, deg_parts, b_stack, g, be, res,
                    wa_stack, ga, bea, weight):
    return pl.pallas_call(
        _post2_att_body,
        grid=(NBLK,),
        in_specs=[
            pl.BlockSpec((NCORES, BN, FD), lambda i: (0, i, 0)),
            pl.BlockSpec((BN, FD), lambda i: (i, 0)),
            pl.BlockSpec((NCORES, BN), lambda i: (0, i)),
            pl.BlockSpec((1, 1, FD), lambda i: (_region(i), 0, 0)),
            pl.BlockSpec((1, BN), lambda i: (0, i)),
            pl.BlockSpec((1, BN), lambda i: (0, i)),
            pl.BlockSpec((BN, FD), lambda i: (i, 0)),
            pl.BlockSpec((1, FD, FD), lambda i: (_region(i), 0, 0)),
            pl.BlockSpec((1, BN), lambda i: (0, i)),
            pl.BlockSpec((1, BN), lambda i: (0, i)),
            pl.BlockSpec((FD, FD), lambda i: (0, 0)),
        ],
        out_specs=[pl.BlockSpec((BN, FD), lambda i: (i, 0)),
                   pl.BlockSpec((BN, FD), lambda i: (i, 0))],
        out_shape=[jax.ShapeDtypeStruct((NC_NODES, FD), jnp.float32),
                   jax.ShapeDtypeStruct((NC_NODES, FD), jnp.float32)],
    )(acc_parts, xs, deg_parts, b_stack, g, be, res, wa_stack, ga, bea, weight)


def _final_body(p_ref, yo_ref, o_ref):
    s = lax.dot_general(p_ref[...], yo_ref[...], (((1,), (1,)), ((), ())),
                        preferred_element_type=jnp.float32)
    o_ref[...] = jax.nn.sigmoid(s)


def _final_call(p_pad, yo):
    return pl.pallas_call(
        _final_body,
        grid=(NLP // BN, ND // BN),
        in_specs=[
            pl.BlockSpec((BN, FD), lambda i, j: (i, 0)),
            pl.BlockSpec((BN, FD), lambda i, j: (j, 0)),
        ],
        out_specs=pl.BlockSpec((BN, BN), lambda i, j: (i, j)),
        out_shape=jax.ShapeDtypeStruct((NLP, ND), jnp.float32),
    )(p_pad, yo)


def _pad_rows(a, n):
    return jnp.concatenate([a, jnp.zeros((n - a.shape[0],) + a.shape[1:],
                                         a.dtype)], axis=0)


def _combine_vec(al, ad):
    return jnp.concatenate([al, jnp.zeros((NLP - NL,), al.dtype), ad]
                           )[None, :]


def kernel(x_l, x_d, lsm_edges, lsm_edge_weight, dsm_edges, dsm_edge_weight,
           W_lsm1, b_lsm1, W_lsm2, b_lsm2, W_dsm1, b_dsm1, W_dsm2, b_dsm2,
           Wa_x, Wa_y, weight,
           g1x, be1x, g2x, be2x, gx, bex,
           g1y, be1y, g2y, be2y, gy, bey):
    npad = EPAD - E
    pad_idx = (jnp.arange(npad, dtype=jnp.int32) % ND)
    row = jnp.concatenate([lsm_edges[0], dsm_edges[0] + NLP, pad_idx])
    col = jnp.concatenate([lsm_edges[1], dsm_edges[1] + NLP, pad_idx])
    col_msg = jnp.concatenate([lsm_edges[1], dsm_edges[1] + NL, pad_idx])
    w = jnp.concatenate([lsm_edge_weight, dsm_edge_weight,
                         jnp.zeros((npad,), jnp.float32)])
    nbt = EPAD // EBLK
    packed = jnp.concatenate(
        [row.reshape(nbt, NCH, CHUNK), col_msg.reshape(nbt, NCH, CHUNK)],
        axis=1)
    w3 = w.reshape(nbt, EBLK)
    xcomb = jnp.concatenate([x_l, jnp.zeros((NLP - NL, FD), jnp.float32),
                             x_d], axis=0)

    W1 = jnp.stack([W_lsm1, W_dsm1])
    W2 = jnp.stack([W_lsm2, W_dsm2])
    Wa = jnp.stack([Wa_x, Wa_y])
    b1 = jnp.stack([b_lsm1, b_dsm1])[:, None, :]
    b2 = jnp.stack([b_lsm2, b_dsm2])[:, None, :]
    g1 = _combine_vec(g1x, g1y)
    be1 = _combine_vec(be1x, be1y)
    g2 = _combine_vec(g2x, g2y)
    be2 = _combine_vec(be2x, be2y)
    ga = _combine_vec(gx, gy)
    bea = _combine_vec(bex, bey)

    deg_parts = _deg_kernel(col, w)

    xs1 = _xs_call(xcomb, W1, deg_parts)
    acc1 = _msg_kernel(xs1, packed, w3)
    x1, xs2 = _post1_xs2_call(acc1, xs1, deg_parts, b1, g1, be1, W2)

    acc2 = _msg_kernel(xs2, packed, w3)
    xo_comb, p = _post2_att_call(acc2, xs2, deg_parts, b2, g2, be2, x1,
                                 Wa, ga, bea, weight)
    yo = xo_comb[NLP:]
    xo = xo_comb[:NL]

    out = _final_call(p[:NLP], yo)[:NL]
    return (xo, yo, out)


# speedup vs baseline: 1.1105x; 1.0185x over previous
import functools

import jax
import jax.numpy as jnp
from jax import lax
from jax.experimental import pallas as pl
from jax.experimental.pallas import tpu as pltpu
from jax.experimental.pallas import tpu_sc as plsc

NL = 10000
ND = 4096
NLP = 10240
NC_NODES = NLP + ND
FD = 128

NCORES = 2
NSUB = 16
NW = NCORES * NSUB

EL = 320000
ED = 131072
E = EL + ED
CHUNK = 48
EBLK = 384
EPT = 14592
EPAD = EPT * NW
NBLKE = EPT // EBLK
NCH = EBLK // CHUNK
NACC = NL + ND
ACC_PER_TILE = 881

ROWS_PER_TILE = NC_NODES // NSUB
BN = 512
NBLK = NC_NODES // BN
LSM_BLOCKS = NLP // BN

_MESH = plsc.VectorSubcoreMesh(core_axis_name="c", subcore_axis_name="s")


def _zero_vmem_2d(ref, nrow):
    def body(j, _):
        for k in range(ref.shape[1] // 16):
            ref[j, pl.ds(k * 16, 16)] = jnp.zeros((16,), jnp.float32)
        return 0
    lax.fori_loop(0, nrow, body, 0)


@functools.partial(
    pl.kernel,
    out_type=jax.ShapeDtypeStruct((NCORES, NC_NODES), jnp.float32),
    mesh=_MESH,
    scratch_types=[
        pltpu.VMEM_SHARED((NC_NODES,), jnp.float32),
        pltpu.VMEM((EBLK,), jnp.int32),
        pltpu.VMEM((EBLK,), jnp.float32),
        pltpu.VMEM((ROWS_PER_TILE,), jnp.float32),
    ],
)
def _deg_kernel(col_hbm, w_hbm, out_hbm, acc, col_v, w_v, zbuf):
    cid = lax.axis_index("c")
    sid = lax.axis_index("s")
    wid = cid * NSUB + sid

    def zb(j, _):
        zbuf[pl.ds(j * 16, 16)] = jnp.zeros((16,), jnp.float32)
        return 0
    lax.fori_loop(0, ROWS_PER_TILE // 16, zb, 0)
    pltpu.sync_copy(zbuf, acc.at[pl.ds(sid * ROWS_PER_TILE, ROWS_PER_TILE)])
    plsc.subcore_barrier()

    def blk_body(t, _):
        base = wid * (EPAD // NW) + t * EBLK
        pltpu.sync_copy(col_hbm.at[pl.ds(base, EBLK)], col_v)
        pltpu.sync_copy(w_hbm.at[pl.ds(base, EBLK)], w_v)
        pltpu.sync_copy(w_v, acc.at[col_v], add=True)
        return 0
    lax.fori_loop(0, (EPAD // NW) // EBLK, blk_body, 0)
    plsc.subcore_barrier()

    pltpu.sync_copy(acc.at[pl.ds(sid * ROWS_PER_TILE, ROWS_PER_TILE)],
                    out_hbm.at[cid, pl.ds(sid * ROWS_PER_TILE, ROWS_PER_TILE)])


@functools.partial(
    pl.kernel,
    out_type=jax.ShapeDtypeStruct((NCORES, NC_NODES, FD), jnp.float32),
    mesh=_MESH,
    scratch_types=[
        pltpu.VMEM_SHARED((NACC, FD), jnp.float32),
        pltpu.VMEM((CHUNK, FD), jnp.float32),
        pltpu.VMEM((CHUNK, FD), jnp.float32),
        pltpu.VMEM((2 * NCH, CHUNK), jnp.int32),
        pltpu.VMEM((2 * NCH, CHUNK), jnp.int32),
        pltpu.VMEM((EBLK,), jnp.float32),
        pltpu.VMEM((EBLK,), jnp.float32),
        pltpu.SemaphoreType.DMA,
        pltpu.SemaphoreType.DMA,
        pltpu.SemaphoreType.DMA,
        pltpu.SemaphoreType.DMA,
        pltpu.SemaphoreType.DMA,
        pltpu.SemaphoreType.DMA,
    ],
)
def _msg_kernel(xs_hbm, packed_hbm, w3_hbm, out_hbm,
                acc, rows_a, rows_b, idx_a, idx_b, wb_a, wb_b,
                gs_a, gs_b, ss_a, ss_b, is_a, is_b):
    cid = lax.axis_index("c")
    sid = lax.axis_index("s")
    wid = cid * NSUB + sid
    bufs = (rows_a, rows_b)
    gsems = (gs_a, gs_b)
    ssems = (ss_a, ss_b)
    ibufs = (idx_a, idx_b)
    wbufs = (wb_a, wb_b)
    isems = (is_a, is_b)

    _zero_vmem_2d(rows_a, CHUNK)
    z0 = sid * ACC_PER_TILE
    done = 0
    while done < ACC_PER_TILE:
        n = min(CHUNK, ACC_PER_TILE - done)
        pltpu.sync_copy(rows_a.at[pl.ds(0, n)], acc.at[pl.ds(z0 + done, n)])
        done += n
    plsc.subcore_barrier()

    def scale(buf, wb, c):
        def scale_grp(jj, _):
            wv = wb[pl.ds(c * CHUNK + jj * 16, 16)]
            for i in range(16):
                wj = wv[i]
                j = jj * 16 + i
                for k in range(FD // 16):
                    buf[j, pl.ds(k * 16, 16)] = buf[j, pl.ds(k * 16, 16)] * wj
            return 0
        lax.fori_loop(0, CHUNK // 16, scale_grp, 0)

    def idx_start(t2, blk, p):
        b = wid * NBLKE + 2 * t2 + blk
        d0 = pltpu.async_copy(packed_hbm.at[b], ibufs[p], isems[p])
        d1 = pltpu.async_copy(w3_hbm.at[b], wbufs[p], isems[p])
        return d0, d1

    def idx_wait(p):
        pltpu.make_async_copy(packed_hbm.at[0], ibufs[p], isems[p]).wait()
        pltpu.make_async_copy(w3_hbm.at[0], wbufs[p], isems[p]).wait()

    def process_pair(t2):
        nc2 = 2 * NCH

        def src(c):
            return ibufs[c // NCH], wbufs[c // NCH], c % NCH

        gd = [None, None]
        sd = [None] * nc2
        ib0, _, _ = src(0)
        gd[0] = pltpu.async_copy(xs_hbm.at[ib0.at[0]], bufs[0], gsems[0])
        for c in range(nc2):
            p = c % 2
            ib, wb, cc = src(c)
            if c + 1 < nc2:
                q = 1 - p
                if c >= 1:
                    sd[c - 1].wait()
                if c + 1 == NCH:
                    idx_wait(1)
                ib1, _, cc1 = src(c + 1)
                gd[q] = pltpu.async_copy(xs_hbm.at[ib1.at[cc1]],
                                         bufs[q], gsems[q])
            gd[p].wait()
            scale(bufs[p], wb, cc)
            sd[c] = pltpu.async_copy(bufs[p], acc.at[ib.at[NCH + cc]],
                                     ssems[p], add=True)
        sd[nc2 - 1].wait()

    for d in idx_start(0, 0, 0):
        d.wait()

    def pair_body(t2, _):
        idx_start(t2, 1, 1)
        process_pair(t2)

        @pl.when(t2 + 1 < NBLKE // 2)
        def _():
            idx_start(t2 + 1, 0, 0)
            idx_wait(0)
        return 0
    lax.fori_loop(0, NBLKE // 2, pair_body, 0)
    plsc.subcore_barrier()

    for s in range(NSUB):
        p0 = s * ROWS_PER_TILE
        p1 = p0 + ROWS_PER_TILE
        if p1 <= NL:
            @pl.when(sid == s)
            def _(p0=p0):
                pltpu.sync_copy(acc.at[pl.ds(p0, ROWS_PER_TILE)],
                                out_hbm.at[cid, pl.ds(p0, ROWS_PER_TILE)])
        elif p0 < NL:
            n1 = NL - p0
            n2 = p1 - NLP
            @pl.when(sid == s)
            def _(p0=p0, n1=n1, n2=n2):
                pltpu.sync_copy(acc.at[pl.ds(p0, n1)],
                                out_hbm.at[cid, pl.ds(p0, n1)])
                pltpu.sync_copy(acc.at[pl.ds(NL, n2)],
                                out_hbm.at[cid, pl.ds(NLP, n2)])
        else:
            @pl.when(sid == s)
            def _(p0=p0):
                pltpu.sync_copy(acc.at[pl.ds(p0 - (NLP - NL), ROWS_PER_TILE)],
                                out_hbm.at[cid, pl.ds(p0, ROWS_PER_TILE)])


def _dinv_block(deg_blk):
    deg = deg_blk[0] + deg_blk[1] + 1.0
    return jnp.where(deg > 0, lax.rsqrt(deg), 0.0)


def _xs_body(x_ref, w_ref, deg_ref, o_ref):
    dinv = _dinv_block(deg_ref[...])
    o_ref[...] = dinv[:, None] * jnp.dot(x_ref[...], w_ref[0],
                                         preferred_element_type=jnp.float32)


def _region(i):
    return jnp.where(i < LSM_BLOCKS, 0, 1)


def _xs_call(x, w_stack, deg_parts):
    return pl.pallas_call(
        _xs_body,
        grid=(NBLK,),
        in_specs=[
            pl.BlockSpec((BN, FD), lambda i: (i, 0)),
            pl.BlockSpec((1, FD, FD), lambda i: (_region(i), 0, 0)),
            pl.BlockSpec((NCORES, BN), lambda i: (0, i)),
        ],
        out_specs=pl.BlockSpec((BN, FD), lambda i: (i, 0)),
        out_shape=jax.ShapeDtypeStruct((NC_NODES, FD), jnp.float32),
    )(x, w_stack, deg_parts)


def _bnorm_relu(t, g, be):
    m = jnp.mean(t, axis=1, keepdims=True)
    v = jnp.mean((t - m) ** 2, axis=1, keepdims=True)
    return jax.nn.relu((t - m) / jnp.sqrt(v + 1e-5) * g[:, None] + be[:, None])


def _post1_xs2_body(acc_ref, xs_ref, deg_ref, b_ref, g_ref, be_ref, w2_ref,
                    x1_ref, xs2_ref):
    dinv = _dinv_block(deg_ref[...])
    t = dinv[:, None] * (acc_ref[0] + acc_ref[1] + xs_ref[...]) + b_ref[0, 0]
    x1 = _bnorm_relu(t, g_ref[0], be_ref[0])
    x1_ref[...] = x1
    xs2_ref[...] = dinv[:, None] * jnp.dot(
        x1, w2_ref[0], preferred_element_type=jnp.float32)


def _post1_xs2_call(acc_parts, xs, deg_parts, b_stack, g, be, w2_stack):
    return pl.pallas_call(
        _post1_xs2_body,
        grid=(NBLK,),
        in_specs=[
            pl.BlockSpec((NCORES, BN, FD), lambda i: (0, i, 0)),
            pl.BlockSpec((BN, FD), lambda i: (i, 0)),
            pl.BlockSpec((NCORES, BN), lambda i: (0, i)),
            pl.BlockSpec((1, 1, FD), lambda i: (_region(i), 0, 0)),
            pl.BlockSpec((1, BN), lambda i: (0, i)),
            pl.BlockSpec((1, BN), lambda i: (0, i)),
            pl.BlockSpec((1, FD, FD), lambda i: (_region(i), 0, 0)),
        ],
        out_specs=[pl.BlockSpec((BN, FD), lambda i: (i, 0)),
                   pl.BlockSpec((BN, FD), lambda i: (i, 0))],
        out_shape=[jax.ShapeDtypeStruct((NC_NODES, FD), jnp.float32),
                   jax.ShapeDtypeStruct((NC_NODES, FD), jnp.float32)],
    )(acc_parts, xs, deg_parts, b_stack, g, be, w2_stack)


def _post2_att_body(acc_ref, xs_ref, deg_ref, b_ref, g_ref, be_ref, res_ref,
                    wa_ref, ga_ref, bea_ref, wt_ref, xo_ref, p_ref):
    dinv = _dinv_block(deg_ref[...])
    t = dinv[:, None] * (acc_ref[0] + acc_ref[1] + xs_ref[...]) + b_ref[0, 0]
    z = _bnorm_relu(t, g_ref[0], be_ref[0])
    x2 = jax.nn.relu(res_ref[...] + z)
    a = jnp.tanh(jnp.dot(x2, wa_ref[0], preferred_element_type=jnp.float32))
    e = jnp.exp(a - jnp.max(a, axis=1, keepdims=True))
    alpha = e / jnp.sum(e, axis=1, keepdims=True)
    xo = _bnorm_relu(alpha * x2, ga_ref[0], bea_ref[0])
    xo_ref[...] = xo
    p_ref[...] = jnp.dot(xo, wt_ref[...], preferred_element_type=jnp.float32)


def _post2_att_call(acc_parts, xs, deg_parts, b_stack, g, be, res,
                    wa_stack, ga, bea, weight):
    return pl.pallas_call(
        _post2_att_body,
        grid=(NBLK,),
        in_specs=[
            pl.BlockSpec((NCORES, BN, FD), lambda i: (0, i, 0)),
            pl.BlockSpec((BN, FD), lambda i: (i, 0)),
            pl.BlockSpec((NCORES, BN), lambda i: (0, i)),
            pl.BlockSpec((1, 1, FD), lambda i: (_region(i), 0, 0)),
            pl.BlockSpec((1, BN), lambda i: (0, i)),
            pl.BlockSpec((1, BN), lambda i: (0, i)),
            pl.BlockSpec((BN, FD), lambda i: (i, 0)),
            pl.BlockSpec((1, FD, FD), lambda i: (_region(i), 0, 0)),
            pl.BlockSpec((1, BN), lambda i: (0, i)),
            pl.BlockSpec((1, BN), lambda i: (0, i)),
            pl.BlockSpec((FD, FD), lambda i: (0, 0)),
        ],
        out_specs=[pl.BlockSpec((BN, FD), lambda i: (i, 0)),
                   pl.BlockSpec((BN, FD), lambda i: (i, 0))],
        out_shape=[jax.ShapeDtypeStruct((NC_NODES, FD), jnp.float32),
                   jax.ShapeDtypeStruct((NC_NODES, FD), jnp.float32)],
    )(acc_parts, xs, deg_parts, b_stack, g, be, res, wa_stack, ga, bea, weight)


def _final_body(p_ref, yo_ref, o_ref):
    s = lax.dot_general(p_ref[...], yo_ref[...], (((1,), (1,)), ((), ())),
                        preferred_element_type=jnp.float32)
    o_ref[...] = jax.nn.sigmoid(s)


def _final_call(p_pad, yo):
    return pl.pallas_call(
        _final_body,
        grid=(NLP // BN, ND // BN),
        in_specs=[
            pl.BlockSpec((BN, FD), lambda i, j: (i, 0)),
            pl.BlockSpec((BN, FD), lambda i, j: (j, 0)),
        ],
        out_specs=pl.BlockSpec((BN, BN), lambda i, j: (i, j)),
        out_shape=jax.ShapeDtypeStruct((NLP, ND), jnp.float32),
    )(p_pad, yo)


def _pad_rows(a, n):
    return jnp.concatenate([a, jnp.zeros((n - a.shape[0],) + a.shape[1:],
                                         a.dtype)], axis=0)


def _combine_vec(al, ad):
    return jnp.concatenate([al, jnp.zeros((NLP - NL,), al.dtype), ad]
                           )[None, :]


def kernel(x_l, x_d, lsm_edges, lsm_edge_weight, dsm_edges, dsm_edge_weight,
           W_lsm1, b_lsm1, W_lsm2, b_lsm2, W_dsm1, b_dsm1, W_dsm2, b_dsm2,
           Wa_x, Wa_y, weight,
           g1x, be1x, g2x, be2x, gx, bex,
           g1y, be1y, g2y, be2y, gy, bey):
    npad = EPAD - E
    pad_idx = (jnp.arange(npad, dtype=jnp.int32) % ND)
    row = jnp.concatenate([lsm_edges[0], dsm_edges[0] + NLP, pad_idx])
    col = jnp.concatenate([lsm_edges[1], dsm_edges[1] + NLP, pad_idx])
    col_msg = jnp.concatenate([lsm_edges[1], dsm_edges[1] + NL, pad_idx])
    w = jnp.concatenate([lsm_edge_weight, dsm_edge_weight,
                         jnp.zeros((npad,), jnp.float32)])
    nbt = EPAD // EBLK
    packed = jnp.concatenate(
        [row.reshape(nbt, NCH, CHUNK), col_msg.reshape(nbt, NCH, CHUNK)],
        axis=1)
    w3 = w.reshape(nbt, EBLK)
    xcomb = jnp.concatenate([x_l, jnp.zeros((NLP - NL, FD), jnp.float32),
                             x_d], axis=0)

    W1 = jnp.stack([W_lsm1, W_dsm1])
    W2 = jnp.stack([W_lsm2, W_dsm2])
    Wa = jnp.stack([Wa_x, Wa_y])
    b1 = jnp.stack([b_lsm1, b_dsm1])[:, None, :]
    b2 = jnp.stack([b_lsm2, b_dsm2])[:, None, :]
    g1 = _combine_vec(g1x, g1y)
    be1 = _combine_vec(be1x, be1y)
    g2 = _combine_vec(g2x, g2y)
    be2 = _combine_vec(be2x, be2y)
    ga = _combine_vec(gx, gy)
    bea = _combine_vec(bex, bey)

    deg_parts = _deg_kernel(col, w)

    xs1 = _xs_call(xcomb, W1, deg_parts)
    acc1 = _msg_kernel(xs1, packed, w3)
    x1, xs2 = _post1_xs2_call(acc1, xs1, deg_parts, b1, g1, be1, W2)

    acc2 = _msg_kernel(xs2, packed, w3)
    xo_comb, p = _post2_att_call(acc2, xs2, deg_parts, b2, g2, be2, x1,
                                 Wa, ga, bea, weight)
    yo = xo_comb[NLP:]
    xo = xo_comb[:NL]

    out = _final_call(p[:NLP], yo)[:NL]
    return (xo, yo, out)
